# baseline (speedup 1.0000x reference)
import functools

import jax
import jax.numpy as jnp
from jax import lax
from jax.experimental import pallas as pl
from jax.experimental.pallas import tpu as pltpu
from jax.experimental.pallas import tpu_sc as plsc

_B = 16384
_D = 128
_NC = 2
_NS = 16
_NW = _NC * _NS
_BPW = _B // _NW
_CH = 128
_NCHUNK = _BPW // _CH
_G = 16
_RPS = _CH // _G

_mesh = plsc.VectorSubcoreMesh(core_axis_name="c", subcore_axis_name="s")


@functools.partial(
    pl.kernel,
    out_type=jax.ShapeDtypeStruct((_B,), jnp.float32),
    mesh=_mesh,
    scratch_types=[
        pltpu.VMEM((_BPW,), jnp.int32),
        pltpu.VMEM((_BPW,), jnp.int32),
        pltpu.VMEM((_BPW,), jnp.int32),
        pltpu.VMEM((_CH, _D), jnp.float32),
        pltpu.VMEM((_CH, _D), jnp.float32),
        pltpu.VMEM((_CH, _D), jnp.float32),
        pltpu.VMEM((_CH, _D), jnp.float32),
        pltpu.VMEM((_CH, _D), jnp.float32),
        pltpu.VMEM((_CH, _D), jnp.float32),
        pltpu.VMEM((_BPW * _G,), jnp.float32),
        pltpu.VMEM((_G, _CH), jnp.int32),
        pltpu.VMEM_SHARED((_NS * _BPW,), jnp.float32),
        pltpu.VMEM((_BPW,), jnp.float32),
        pltpu.SemaphoreType.DMA,
        pltpu.SemaphoreType.DMA,
        pltpu.SemaphoreType.DMA,
        pltpu.SemaphoreType.DMA,
        pltpu.SemaphoreType.DMA,
        pltpu.SemaphoreType.DMA,
        pltpu.SemaphoreType.DMA,
    ],
)
def _distmult(heads_hbm, rels_hbm, tails_hbm, etab_hbm, rtab_hbm,
              out_hbm, hidx, ridx, tidx,
              hb0, rb0, tb0, hb1, rb1, tb1, pbuf, sidx, sbuf, zbuf,
              hs0, rs0, ts0, hs1, rs1, ts1, asem):
    sid = lax.axis_index("s")
    wid = sid * _NC + lax.axis_index("c")
    base = wid * _BPW
    soff = sid * _BPW
    pltpu.sync_copy(heads_hbm.at[pl.ds(base, _BPW)], hidx)
    pltpu.sync_copy(rels_hbm.at[pl.ds(base, _BPW)], ridx)
    pltpu.sync_copy(tails_hbm.at[pl.ds(base, _BPW)], tidx)

    def start(c, hb, rb, tb, hs, rs, ts):
        cb = c * _CH
        pltpu.async_copy(etab_hbm.at[hidx.at[pl.ds(cb, _CH)]], hb, hs)
        pltpu.async_copy(rtab_hbm.at[ridx.at[pl.ds(cb, _CH)]], rb, rs)
        pltpu.async_copy(etab_hbm.at[tidx.at[pl.ds(cb, _CH)]], tb, ts)

    set0 = (hb0, rb0, tb0, hs0, rs0, ts0)
    set1 = (hb1, rb1, tb1, hs1, rs1, ts1)
    start(0, *set0)
    start(1, *set1)

    zeros = jnp.zeros((16,), jnp.float32)
    for m in range(_BPW // _G):
        zbuf[pl.ds(m * _G, _G)] = zeros
    pltpu.sync_copy(zbuf, sbuf.at[pl.ds(soff, _BPW)])
    for m in range(_G):
        for seg in range(_RPS):
            sidx[m, pl.ds(seg * _G, _G)] = jnp.full((16,), m * _RPS + seg,
                                                    jnp.int32)

    def wait(hb, rb, tb, hs, rs, ts):
        dummy = etab_hbm.at[pl.ds(0, _CH)]
        pltpu.make_async_copy(dummy, hb, hs).wait()
        pltpu.make_async_copy(dummy, rb, rs).wait()
        pltpu.make_async_copy(dummy, tb, ts).wait()

    def compute(c, hb, rb, tb):
        cb = c * _CH

        @plsc.parallel_loop(0, _CH, step=1, unroll=16)
        def _row(row):
            acc = jnp.zeros((16,), jnp.float32)
            for s in range(_D // 16):
                sl = pl.ds(s * 16, 16)
                acc = acc + hb[row, sl] * rb[row, sl] * tb[row, sl]
            pbuf[pl.ds((cb + row) * _G, _G)] = acc

        for m in range(_G):
            src = pbuf.at[pl.ds((cb + m * _RPS) * _G, _CH)]
            dst = sbuf.at[pl.ds(soff + cb, _CH)]
            pltpu.async_copy(src, dst.at[sidx.at[m]], asem, add=True)

    def drain():
        for _ in range(_G * _NCHUNK):
            pltpu.make_async_copy(pbuf.at[pl.ds(0, _CH)],
                                  sbuf.at[pl.ds(soff, _CH)].at[sidx.at[0]],
                                  asem).wait()

    def pair(p, carry):
        c0 = 2 * p
        wait(*set0)
        compute(c0, hb0, rb0, tb0)

        @pl.when(p < _NCHUNK // 2 - 1)
        def _():
            start(c0 + 2, *set0)

        wait(*set1)
        compute(c0 + 1, hb1, rb1, tb1)

        @pl.when(p < _NCHUNK // 2 - 1)
        def _():
            start(c0 + 3, *set1)

        return carry

    lax.fori_loop(0, _NCHUNK // 2, pair, 0)
    drain()
    pltpu.sync_copy(sbuf.at[pl.ds(soff, _BPW)], out_hbm.at[pl.ds(base, _BPW)])


def kernel(heads, rels, tails, entity_table, relation_table):
    return _distmult(heads, rels, tails, entity_table, relation_table)


# speedup vs baseline: 1.2798x; 1.2798x over previous
import functools

import jax
import jax.numpy as jnp
from jax import lax
from jax.experimental import pallas as pl
from jax.experimental.pallas import tpu as pltpu
from jax.experimental.pallas import tpu_sc as plsc

_B = 16384
_D = 128
_NC = 2
_NS = 16
_NW = _NC * _NS
_BPW = _B // _NW
_CH = 128
_NCHUNK = _BPW // _CH
_G = 16
_RPS = _CH // _G

_mesh = plsc.VectorSubcoreMesh(core_axis_name="c", subcore_axis_name="s")


@functools.partial(
    pl.kernel,
    out_type=jax.ShapeDtypeStruct((_B,), jnp.float32),
    mesh=_mesh,
    scratch_types=[
        pltpu.VMEM((_BPW,), jnp.int32),
        pltpu.VMEM((_BPW,), jnp.int32),
        pltpu.VMEM((_BPW,), jnp.int32),
        pltpu.VMEM((_CH, _D), jnp.float32),
        pltpu.VMEM((_CH, _D), jnp.float32),
        pltpu.VMEM((_CH, _D), jnp.float32),
        pltpu.VMEM((_CH, _D), jnp.float32),
        pltpu.VMEM((_CH, _D), jnp.float32),
        pltpu.VMEM((_CH, _D), jnp.float32),
        pltpu.VMEM((_BPW * _G,), jnp.float32),
        pltpu.VMEM((_G, _CH), jnp.int32),
        pltpu.VMEM_SHARED((_NS * _BPW,), jnp.float32),
        pltpu.VMEM((_BPW,), jnp.float32),
        pltpu.SemaphoreType.DMA,
        pltpu.SemaphoreType.DMA,
        pltpu.SemaphoreType.DMA,
        pltpu.SemaphoreType.DMA,
        pltpu.SemaphoreType.DMA,
        pltpu.SemaphoreType.DMA,
        pltpu.SemaphoreType.DMA,
    ],
)
def _distmult(heads_hbm, rels_hbm, tails_hbm, etab_hbm, rtab_hbm,
              out_hbm, hidx, ridx, tidx,
              hb0, rb0, tb0, hb1, rb1, tb1, pbuf, sidx, sbuf, zbuf,
              hs0, rs0, ts0, hs1, rs1, ts1, asem):
    sid = lax.axis_index("s")
    wid = sid * _NC + lax.axis_index("c")
    base = wid * _BPW
    soff = sid * _BPW
    pltpu.sync_copy(heads_hbm.at[pl.ds(base, _BPW)], hidx)
    pltpu.sync_copy(rels_hbm.at[pl.ds(base, _BPW)], ridx)
    pltpu.sync_copy(tails_hbm.at[pl.ds(base, _BPW)], tidx)

    zeros = jnp.zeros((16,), jnp.float32)
    for m in range(_BPW // _G):
        zbuf[pl.ds(m * _G, _G)] = zeros
    pltpu.sync_copy(zbuf, sbuf.at[pl.ds(soff, _BPW)])
    for m in range(_G):
        for seg in range(_RPS):
            sidx[m, pl.ds(seg * _G, _G)] = jnp.full((16,), m * _RPS + seg,
                                                    jnp.int32)

    def start(c, hb, rb, tb, hs, rs, ts):
        cb = c * _CH
        pltpu.async_copy(etab_hbm.at[hidx.at[pl.ds(cb, _CH)]], hb, hs)
        pltpu.async_copy(rtab_hbm.at[ridx.at[pl.ds(cb, _CH)]], rb, rs)
        pltpu.async_copy(etab_hbm.at[tidx.at[pl.ds(cb, _CH)]], tb, ts)

    def wait(hb, rb, tb, hs, rs, ts):
        dummy = etab_hbm.at[pl.ds(0, _CH)]
        pltpu.make_async_copy(dummy, hb, hs).wait()
        pltpu.make_async_copy(dummy, rb, rs).wait()
        pltpu.make_async_copy(dummy, tb, ts).wait()

    def compute(c, hb, rb, tb):
        cb = c * _CH

        @plsc.parallel_loop(0, _CH, step=1, unroll=8)
        def _row(row):
            acc = jnp.zeros((16,), jnp.float32)
            for s in range(_D // 16):
                sl = pl.ds(s * 16, 16)
                acc = acc + hb[row, sl] * rb[row, sl] * tb[row, sl]
            pbuf[pl.ds((cb + row) * _G, _G)] = acc

        for m in range(_G):
            src = pbuf.at[pl.ds((cb + m * _RPS) * _G, _CH)]
            dst = sbuf.at[pl.ds(soff + cb, _CH)]
            pltpu.async_copy(src, dst.at[sidx.at[m]], asem, add=True)

    def drain():
        for _ in range(_G * _NCHUNK):
            pltpu.make_async_copy(pbuf.at[pl.ds(0, _CH)],
                                  sbuf.at[pl.ds(soff, _CH)].at[sidx.at[0]],
                                  asem).wait()

    set0 = (hb0, rb0, tb0, hs0, rs0, ts0)
    set1 = (hb1, rb1, tb1, hs1, rs1, ts1)
    start(0, *set0)

    def pair(p, carry):
        c0 = 2 * p
        start(c0 + 1, *set1)
        wait(*set0)
        compute(c0, hb0, rb0, tb0)

        @pl.when(p < _NCHUNK // 2 - 1)
        def _():
            start(c0 + 2, *set0)

        wait(*set1)
        compute(c0 + 1, hb1, rb1, tb1)
        return carry

    lax.fori_loop(0, _NCHUNK // 2, pair, 0)
    drain()
    pltpu.sync_copy(sbuf.at[pl.ds(soff, _BPW)], out_hbm.at[pl.ds(base, _BPW)])


def kernel(heads, rels, tails, entity_table, relation_table):
    return _distmult(heads, rels, tails, entity_table, relation_table)
